# baseline (speedup 1.0000x reference)
import functools

import jax
import jax.numpy as jnp
from jax import lax
from jax.experimental import pallas as pl
from jax.experimental.pallas import tpu as pltpu
from jax.experimental.pallas import tpu_sc as plsc


def _make_sc_gather(B, D, CH):
    info = plsc.get_sparse_core_info()
    NC, NS = info.num_cores, info.num_subcores
    NW = NC * NS
    assert B % NW == 0
    b_per_w = B // NW
    assert b_per_w % CH == 0
    n_chunks = b_per_w // CH

    mesh = plsc.VectorSubcoreMesh(core_axis_name="c", subcore_axis_name="s")

    @functools.partial(
        pl.kernel,
        mesh=mesh,
        out_type=jax.ShapeDtypeStruct((B, D), jnp.float32),
        scratch_types=[
            pltpu.VMEM((CH,), jnp.int32),
            pltpu.VMEM((CH, D), jnp.float32),
            pltpu.SemaphoreType.DMA,
        ],
        compiler_params=pltpu.CompilerParams(use_tc_tiling_on_sc=False),
    )
    def k(idx_hbm, table_hbm, out_hbm, idx_v, rows_v, sem):
        wid = lax.axis_index("s") * NC + lax.axis_index("c")
        base = wid * b_per_w

        def body(g, carry):
            off = base + g * CH
            pltpu.sync_copy(idx_hbm.at[pl.ds(off, CH)], idx_v)
            pltpu.async_copy(table_hbm.at[idx_v], rows_v, sem).wait()
            pltpu.sync_copy(rows_v, out_hbm.at[pl.ds(off, CH)])
            return carry

        lax.fori_loop(0, n_chunks, body, 0)

    return k


def kernel(x, table):
    Br, S = x.shape
    _, D = table.shape
    B = Br * S
    xf = x.reshape(B)
    out = _make_sc_gather(B, D, 1024)(xf, table)
    return out.reshape(Br, S, D)


# speedup vs baseline: 1.0312x; 1.0312x over previous
import functools

import jax
import jax.numpy as jnp
from jax import lax
from jax.experimental import pallas as pl
from jax.experimental.pallas import tpu as pltpu
from jax.experimental.pallas import tpu_sc as plsc


def _make_sc_gather(B, D, CH):
    info = plsc.get_sparse_core_info()
    NC, NS = info.num_cores, info.num_subcores
    NW = NC * NS
    assert B % NW == 0
    b_per_w = B // NW
    assert b_per_w % (2 * CH) == 0
    n_chunks = b_per_w // CH
    n_outer = n_chunks // 2

    mesh = plsc.VectorSubcoreMesh(core_axis_name="c", subcore_axis_name="s")

    @functools.partial(
        pl.kernel,
        mesh=mesh,
        out_type=jax.ShapeDtypeStruct((B, D), jnp.float32),
        scratch_types=[
            pltpu.VMEM((CH,), jnp.int32),
            pltpu.VMEM((CH,), jnp.int32),
            pltpu.VMEM((CH, D), jnp.float32),
            pltpu.VMEM((CH, D), jnp.float32),
            pltpu.SemaphoreType.DMA,
            pltpu.SemaphoreType.DMA,
            pltpu.SemaphoreType.DMA,
            pltpu.SemaphoreType.DMA,
        ],
        compiler_params=pltpu.CompilerParams(use_tc_tiling_on_sc=False),
    )
    def k(idx_hbm, table_hbm, out_hbm, i0, i1, r0, r1, sg0, sg1, sw0, sw1):
        idx_v = (i0, i1)
        rows_v = (r0, r1)
        sg = (sg0, sg1)
        sw = (sw0, sw1)
        wid = lax.axis_index("s") * NC + lax.axis_index("c")
        base = wid * b_per_w

        for b in range(2):
            off = base + b * CH
            pltpu.sync_copy(idx_hbm.at[pl.ds(off, CH)], idx_v[b])
            pltpu.async_copy(table_hbm.at[idx_v[b]], rows_v[b], sg[b])

        def step(i, b, prefetch):
            off = base + i * CH
            pltpu.make_async_copy(table_hbm.at[idx_v[b]], rows_v[b], sg[b]).wait()
            pltpu.async_copy(rows_v[b], out_hbm.at[pl.ds(off, CH)], sw[b])
            if prefetch:
                off2 = off + 2 * CH
                pltpu.sync_copy(idx_hbm.at[pl.ds(off2, CH)], idx_v[b])
                pltpu.make_async_copy(
                    rows_v[b], out_hbm.at[pl.ds(off, CH)], sw[b]
                ).wait()
                pltpu.async_copy(table_hbm.at[idx_v[b]], rows_v[b], sg[b])

        def outer(j, carry):
            for b in range(2):
                step(2 * j + b, b, prefetch=True)
            return carry

        lax.fori_loop(0, n_outer - 1, outer, 0)
        for b in range(2):
            i = n_chunks - 2 + b
            step(i, b, prefetch=False)
            pltpu.make_async_copy(
                rows_v[b], out_hbm.at[pl.ds(base + i * CH, CH)], sw[b]
            ).wait()

    return k


def kernel(x, table):
    Br, S = x.shape
    _, D = table.shape
    B = Br * S
    xf = x.reshape(B)
    out = _make_sc_gather(B, D, 800)(xf, table)
    return out.reshape(Br, S, D)
